# baseline (speedup 1.0000x reference)
import functools

import jax
import jax.numpy as jnp
from jax.experimental import pallas as pl
from jax.experimental.pallas import tpu as pltpu

_DN = (((1,), (1,)), ((), ()))


def _body(keym_ref, keyn_ref, xrow_ref, w0_ref, w1_ref, bias_ref, x_hbm_ref,
          y_ref, xblk_ref, sem, *, bm, bn, nblks, mpb):
    r = pl.program_id(0)
    b = r // mpb
    mi = jax.lax.rem(r, mpb)

    y_ref[...] = jax.lax.dot_general(
        xrow_ref[0], w1_ref[...], _DN, preferred_element_type=jnp.float32
    )[None] + bias_ref[...]

    keym = keym_ref[0]
    m_glob = mi * bm + jax.lax.broadcasted_iota(jnp.int32, (bm, 1), 0)

    def mask_block(nb):
        keyn = keyn_ref[0, :, nb * bn:(nb + 1) * bn]
        n_glob = nb * bn + jax.lax.broadcasted_iota(jnp.int32, (bm, bn), 1)
        return ((keym == keyn) & (n_glob < m_glob)).astype(jnp.float32)

    rowsums = [jnp.sum(mask_block(nb), axis=1, keepdims=True)
               for nb in range(nblks)]
    cnt = sum(rowsums)

    @pl.when(jnp.sum(cnt) > 0)
    def _correct():
        scale = jnp.where(cnt > 0, 1.0 / jnp.where(cnt > 0, cnt, 1.0), 0.0)
        for nb in range(nblks):

            @pl.when(jnp.sum(rowsums[nb]) > 0)
            def _acc(nb=nb):
                cp = pltpu.make_async_copy(
                    x_hbm_ref.at[b, pl.ds(nb * bn, bn), :], xblk_ref, sem)
                cp.start()
                cp.wait()
                z0b = jax.lax.dot_general(
                    xblk_ref[...], w0_ref[...], _DN,
                    preferred_element_type=jnp.float32)
                y_ref[...] += jnp.dot(mask_block(nb) * scale, z0b,
                                      preferred_element_type=jnp.float32)[None]


def kernel(x, input_ids, W0, b0, W1, b1):
    B, S, D = x.shape
    bm, bn = 512, 256
    mpb = S // bm
    nblks = S // bn
    R = B * S

    ids = input_ids.astype(jnp.int32)
    key = ids[:, :-1] * 1000 + ids[:, 1:]
    keyM = jnp.concatenate(
        [jnp.full((B, 1), -1, jnp.int32), key], axis=1)
    keyN = jnp.concatenate(
        [jnp.full((B, 2), -2, jnp.int32), key[:, :-1]], axis=1)
    bias = (b0 + b1).reshape(1, D)

    y = pl.pallas_call(
        functools.partial(_body, bm=bm, bn=bn, nblks=nblks, mpb=mpb),
        grid=(R // bm,),
        in_specs=[
            pl.BlockSpec((1, bm, 1),
                         lambda r: (r // mpb, jax.lax.rem(r, mpb), 0)),
            pl.BlockSpec((1, 1, S), lambda r: (r // mpb, 0, 0)),
            pl.BlockSpec((1, bm, D),
                         lambda r: (r // mpb, jax.lax.rem(r, mpb), 0)),
            pl.BlockSpec((D, D), lambda r: (0, 0)),
            pl.BlockSpec((D, D), lambda r: (0, 0)),
            pl.BlockSpec((1, D), lambda r: (0, 0)),
            pl.BlockSpec(memory_space=pltpu.MemorySpace.HBM),
        ],
        out_specs=pl.BlockSpec((1, bm, D),
                               lambda r: (r // mpb, jax.lax.rem(r, mpb), 0)),
        out_shape=jax.ShapeDtypeStruct((B, S, D), jnp.float32),
        scratch_shapes=[
            pltpu.VMEM((bn, D), jnp.float32),
            pltpu.SemaphoreType.DMA,
        ],
        compiler_params=pltpu.CompilerParams(
            dimension_semantics=("arbitrary",),
            vmem_limit_bytes=62 * 1024 * 1024),
    )(keyM[:, :, None], keyN[:, None, :], x, W0, W1, bias, x)
    return y


# speedup vs baseline: 1.0003x; 1.0003x over previous
import functools

import jax
import jax.numpy as jnp
from jax.experimental import pallas as pl
from jax.experimental.pallas import tpu as pltpu

_DN = (((1,), (1,)), ((), ()))


def _body(keym_ref, keyn_ref, xrow_ref, w0_ref, w1_ref, bias_ref, x_hbm_ref,
          y_ref, xblk_ref, sem, *, bm, bn, nblks, mpb):
    r = pl.program_id(0)
    b = r // mpb
    mi = jax.lax.rem(r, mpb)

    y_ref[...] = jax.lax.dot_general(
        xrow_ref[...], w1_ref[...], _DN, preferred_element_type=jnp.float32
    ) + bias_ref[...]

    keym = keym_ref[...]
    m_glob = mi * bm + jax.lax.broadcasted_iota(jnp.int32, (bm, 1), 0)

    def mask_block(nb):
        keyn = keyn_ref[0, :, nb * bn:(nb + 1) * bn]
        n_glob = nb * bn + jax.lax.broadcasted_iota(jnp.int32, (bm, bn), 1)
        return ((keym == keyn) & (n_glob < m_glob)).astype(jnp.float32)

    rowsums = [jnp.sum(mask_block(nb), axis=1, keepdims=True)
               for nb in range(nblks)]
    cnt = sum(rowsums)

    @pl.when(jnp.sum(cnt) > 0)
    def _correct():
        scale = jnp.where(cnt > 0, 1.0 / jnp.where(cnt > 0, cnt, 1.0), 0.0)
        for nb in range(nblks):

            @pl.when(jnp.sum(rowsums[nb]) > 0)
            def _acc(nb=nb):
                cp = pltpu.make_async_copy(
                    x_hbm_ref.at[b, pl.ds(nb * bn, bn), :], xblk_ref, sem)
                cp.start()
                cp.wait()
                z0b = jax.lax.dot_general(
                    xblk_ref[...], w0_ref[...], _DN,
                    preferred_element_type=jnp.float32)
                y_ref[...] += jnp.dot(mask_block(nb) * scale, z0b,
                                      preferred_element_type=jnp.float32)


def kernel(x, input_ids, W0, b0, W1, b1):
    B, S, D = x.shape
    bm, bn = 512, 256
    mpb = S // bm
    nblks = S // bn
    R = B * S

    ids = input_ids.astype(jnp.int32)
    key = ids[:, :-1] * 1000 + ids[:, 1:]
    keyM = jnp.concatenate(
        [jnp.full((B, 1), -1, jnp.int32), key], axis=1)
    keyN = jnp.concatenate(
        [jnp.full((B, 2), -2, jnp.int32), key[:, :-1]], axis=1)
    bias = (b0 + b1).reshape(1, D)

    y = pl.pallas_call(
        functools.partial(_body, bm=bm, bn=bn, nblks=nblks, mpb=mpb),
        grid=(R // bm,),
        in_specs=[
            pl.BlockSpec((bm, 1), lambda r: (r, 0)),
            pl.BlockSpec((1, 1, S), lambda r: (r // mpb, 0, 0)),
            pl.BlockSpec((bm, D), lambda r: (r, 0)),
            pl.BlockSpec((D, D), lambda r: (0, 0)),
            pl.BlockSpec((D, D), lambda r: (0, 0)),
            pl.BlockSpec((1, D), lambda r: (0, 0)),
            pl.BlockSpec(memory_space=pltpu.MemorySpace.HBM),
        ],
        out_specs=pl.BlockSpec((bm, D), lambda r: (r, 0)),
        out_shape=jax.ShapeDtypeStruct((R, D), jnp.float32),
        scratch_shapes=[
            pltpu.VMEM((bn, D), jnp.float32),
            pltpu.SemaphoreType.DMA,
        ],
        compiler_params=pltpu.CompilerParams(
            dimension_semantics=("arbitrary",),
            vmem_limit_bytes=62 * 1024 * 1024),
    )(keyM.reshape(R, 1), keyN[:, None, :], x.reshape(R, D), W0, W1, bias, x)
    return y.reshape(B, S, D)
